# Initial kernel scaffold; baseline (speedup 1.0000x reference)
import jax
import jax.numpy as jnp
from jax.experimental import pallas as pl


def kernel(x, edge_index, W1, b1, g1, bt1, W2, b2, g2, bt2, W3, b3):
    raise NotImplementedError("write your pallas kernel here")


# speedup vs baseline: 13.7844x; 13.7844x over previous
import functools

import jax
import jax.numpy as jnp
from jax import lax
from jax.experimental import pallas as pl
from jax.experimental.pallas import tpu as pltpu
from jax.experimental.pallas import tpu_sc as plsc

_N = 10000
_E = 320000
_D = 128
_DOUT = 64
_EPS = 1e-5

_NC = 2
_NS = 16
_NW = _NC * _NS
_EPW = _E // _NW
_EB = 80
_NB = _EPW // _EB
_ZR = 125
_RPT = _N // _NS
_VCH = 1000
_NVT = _N // _VCH

_mesh = plsc.VectorSubcoreMesh(core_axis_name="c", subcore_axis_name="s")


@functools.partial(
    pl.kernel,
    mesh=_mesh,
    out_type=jax.ShapeDtypeStruct((_NC * _N,), jnp.float32),
    scratch_types=[
        pltpu.VMEM((_EB,), jnp.int32),
        pltpu.VMEM((_EB,), jnp.float32),
        pltpu.VMEM((_VCH,), jnp.float32),
        pltpu.VMEM_SHARED((_N,), jnp.float32),
    ],
)
def _sc_deg(dst_h, z1_h, out_h, didx, ones, vstage, deg_sh):
    c = lax.axis_index("c")
    s = lax.axis_index("s")
    wid = c * _NS + s
    for k in range(_EB // 16):
        ones[pl.ds(16 * k, 16)] = jnp.full((16,), 1.0, jnp.float32)

    @pl.when(s < _NVT)
    def _zero():
        pltpu.sync_copy(z1_h, vstage)
        pltpu.sync_copy(vstage, deg_sh.at[pl.ds(s * _VCH, _VCH)])

    plsc.subcore_barrier()

    eoff = wid * _EPW

    def step(i, carry):
        pltpu.sync_copy(dst_h.at[pl.ds(eoff + i * _EB, _EB)], didx)
        pltpu.sync_copy(ones, deg_sh.at[didx], add=True)
        return carry

    lax.fori_loop(0, _NB, step, 0)
    plsc.subcore_barrier()

    @pl.when(s < _NVT)
    def _writeback():
        pltpu.sync_copy(deg_sh.at[pl.ds(s * _VCH, _VCH)], vstage)
        pltpu.sync_copy(vstage, out_h.at[pl.ds(c * _N + s * _VCH, _VCH)])


_SR = 200
_NSH = _VCH // _SR


def _make_sc_agg(with_s):
    out_type = [jax.ShapeDtypeStruct((_NC * _N, _D), jnp.float32)]
    scratch = [
        pltpu.VMEM((_EB,), jnp.int32),
        pltpu.VMEM((_EB,), jnp.int32),
        pltpu.VMEM((_EB, _D), jnp.float32),
        pltpu.VMEM((_SR, _D), jnp.float32),
        pltpu.VMEM_SHARED((_N, _D), jnp.float32),
        pltpu.SemaphoreType.DMA,
    ]
    if with_s:
        out_type.append(jax.ShapeDtypeStruct((_NC * _N,), jnp.float32))
        scratch += [
            pltpu.VMEM((_EB,), jnp.float32),
            pltpu.VMEM((_VCH,), jnp.float32),
            pltpu.VMEM_SHARED((_N,), jnp.float32),
        ]

    def body(src_h, dst_h, y_h, dinv_h, z2_h, z1_h, out_h, *rest):
        if with_s:
            sout_h, sidx, didx, rows, stage, acc_sh, sem, dvals, vstage, s_sh = rest
        else:
            sidx, didx, rows, stage, acc_sh, sem = rest
        c = lax.axis_index("c")
        s = lax.axis_index("s")
        wid = c * _NS + s

        pltpu.sync_copy(z2_h, stage)

        @pl.when(s < _NVT)
        def _zero():
            for j in range(_NSH):
                pltpu.sync_copy(
                    stage, acc_sh.at[pl.ds(s * _VCH + j * _SR, _SR)])
            if with_s:
                pltpu.sync_copy(z1_h, vstage)
                pltpu.sync_copy(vstage, s_sh.at[pl.ds(s * _VCH, _VCH)])
        plsc.subcore_barrier()

        eoff = wid * _EPW

        def step(i, carry):
            b = eoff + i * _EB
            pltpu.sync_copy(src_h.at[pl.ds(b, _EB)], sidx)
            pltpu.sync_copy(dst_h.at[pl.ds(b, _EB)], didx)
            pltpu.async_copy(y_h.at[sidx], rows, sem).wait()
            pltpu.sync_copy(rows, acc_sh.at[didx], add=True)
            if with_s:
                pltpu.async_copy(dinv_h.at[didx], dvals, sem).wait()
                pltpu.sync_copy(dvals, s_sh.at[sidx], add=True)
            return carry

        lax.fori_loop(0, _NB, step, 0)
        plsc.subcore_barrier()

        @pl.when(s < _NVT)
        def _writeback():
            for j in range(_NSH):
                r0 = s * _VCH + j * _SR
                pltpu.sync_copy(acc_sh.at[pl.ds(r0, _SR)], stage)
                pltpu.sync_copy(stage, out_h.at[pl.ds(c * _N + r0, _SR)])
            if with_s:
                pltpu.sync_copy(s_sh.at[pl.ds(s * _VCH, _VCH)], vstage)
                pltpu.sync_copy(vstage, sout_h.at[pl.ds(c * _N + s * _VCH, _VCH)])

    return pl.kernel(body, mesh=_mesh, out_type=out_type, scratch_types=scratch)


_sc_agg_s = _make_sc_agg(True)
_sc_agg = _make_sc_agg(False)


_RB = 1000
_NRB = _N // _RB


def _tc1_body(degp_ref, x_ref, w_ref, dinv_ref, y_ref):
    deg = degp_ref[0, :, 0] + degp_ref[1, :, 0] + 1.0
    dv = lax.rsqrt(deg)
    dinv_ref[...] = dv[:, None]
    y_ref[...] = dv[:, None] * jnp.dot(
        x_ref[...], w_ref[...], preferred_element_type=jnp.float32)


def _tc1(degp, x, W1):
    return pl.pallas_call(
        _tc1_body,
        grid=(_NRB,),
        in_specs=[
            pl.BlockSpec((2, _RB, 1), lambda i: (0, i, 0)),
            pl.BlockSpec((_RB, _D), lambda i: (i, 0)),
            pl.BlockSpec((_D, _D), lambda i: (0, 0)),
        ],
        out_specs=[
            pl.BlockSpec((_RB, 1), lambda i: (i, 0)),
            pl.BlockSpec((_RB, _D), lambda i: (i, 0)),
        ],
        out_shape=[
            jax.ShapeDtypeStruct((_N, 1), jnp.float32),
            jax.ShapeDtypeStruct((_N, _D), jnp.float32),
        ],
    )(degp[:, :, None], x, W1)


def _tc2_body(p_ref, y1_ref, dinv_ref, w2_ref, b1_ref, g1_ref, bt1_ref, y2_ref):
    dv = dinv_ref[...]
    agg = p_ref[0] + p_ref[1] + y1_ref[...]
    gscale = g1_ref[...] * (1.0 / jnp.sqrt(1.0 + _EPS))
    h1 = jnp.maximum((dv * agg + b1_ref[...]) * gscale + bt1_ref[...], 0.0)
    y2_ref[...] = dv * jnp.dot(
        h1, w2_ref[...], preferred_element_type=jnp.float32)


def _tc2(p, y1, dinv, W2, b1, g1, bt1):
    vec = pl.BlockSpec((1, _D), lambda i: (0, 0))
    return pl.pallas_call(
        _tc2_body,
        grid=(_NRB,),
        in_specs=[
            pl.BlockSpec((2, _RB, _D), lambda i: (0, i, 0)),
            pl.BlockSpec((_RB, _D), lambda i: (i, 0)),
            pl.BlockSpec((_RB, 1), lambda i: (i, 0)),
            pl.BlockSpec((_D, _D), lambda i: (0, 0)),
            vec, vec, vec,
        ],
        out_specs=pl.BlockSpec((_RB, _D), lambda i: (i, 0)),
        out_shape=jax.ShapeDtypeStruct((_N, _D), jnp.float32),
    )(p, y1, dinv, W2, b1[None, :], g1[None, :], bt1[None, :])


def _tc3_body(q_ref, y2_ref, dinv_ref, sp_ref, w3_ref, b2_ref, g2_ref,
              bt2_ref, b3_ref, out_ref, acc_ref):
    i = pl.program_id(0)
    dv = dinv_ref[...]
    agg = q_ref[0] + q_ref[1] + y2_ref[...]
    gscale = g2_ref[...] * (1.0 / jnp.sqrt(1.0 + _EPS))
    h2 = jnp.maximum((dv * agg + b2_ref[...]) * gscale + bt2_ref[...], 0.0)
    sv = sp_ref[0, :, 0] + sp_ref[1, :, 0]
    wv = dv[:, 0] * (dv[:, 0] + sv)
    part = jnp.dot(wv[None, :], h2, preferred_element_type=jnp.float32)

    @pl.when(i == 0)
    def _init():
        acc_ref[...] = jnp.zeros_like(acc_ref)

    acc_ref[...] += part

    @pl.when(i == pl.num_programs(0) - 1)
    def _fin():
        out_ref[...] = (
            jnp.dot(acc_ref[...], w3_ref[...],
                    preferred_element_type=jnp.float32) / _N + b3_ref[...])


def _tc3(q, y2, dinv, sp, W3, b2, g2, bt2, b3):
    vec = pl.BlockSpec((1, _D), lambda i: (0, 0))
    return pl.pallas_call(
        _tc3_body,
        grid=(_NRB,),
        in_specs=[
            pl.BlockSpec((2, _RB, _D), lambda i: (0, i, 0)),
            pl.BlockSpec((_RB, _D), lambda i: (i, 0)),
            pl.BlockSpec((_RB, 1), lambda i: (i, 0)),
            pl.BlockSpec((2, _RB, 1), lambda i: (0, i, 0)),
            pl.BlockSpec((_D, _DOUT), lambda i: (0, 0)),
            vec, vec, vec,
            pl.BlockSpec((1, _DOUT), lambda i: (0, 0)),
        ],
        out_specs=pl.BlockSpec((1, _DOUT), lambda i: (0, 0)),
        out_shape=jax.ShapeDtypeStruct((1, _DOUT), jnp.float32),
        scratch_shapes=[pltpu.VMEM((1, _D), jnp.float32)],
    )(q, y2, dinv, sp[:, :, None], W3, b2[None, :], g2[None, :], bt2[None, :],
      b3[None, :])


def kernel(x, edge_index, W1, b1, g1, bt1, W2, b2, g2, bt2, W3, b3):
    src = edge_index[0]
    dst = edge_index[1]
    z1 = jnp.zeros((_VCH,), jnp.float32)
    z2 = jnp.zeros((_SR, _D), jnp.float32)

    degp = _sc_deg(dst, z1).reshape(_NC, _N)
    dinv, y1 = _tc1(degp, x, W1)
    p, sp = _sc_agg_s(src, dst, y1, dinv.reshape(_N), z2, z1)
    p = p.reshape(_NC, _N, _D)
    sp = sp.reshape(_NC, _N)
    y2 = _tc2(p, y1, dinv, W2, b1, g1, bt1)
    (q,) = _sc_agg(src, dst, y2, dinv.reshape(_N), z2, z1)
    q = q.reshape(_NC, _N, _D)
    out = _tc3(q, y2, dinv, sp, W3, b2, g2, bt2, b3)
    return out
